# baseline (speedup 1.0000x reference)
import functools

import jax
import jax.numpy as jnp
from jax import lax
from jax.experimental import pallas as pl
from jax.experimental.pallas import tpu as pltpu
from jax.experimental.pallas import tpu_sc as plsc

N = 10000
E = 320000
D = 128
H = 128

NC = 2
NS = 16
NW = NC * NS

NP = 10240
RB = 128
NB = NP // RB
EPW = E // NW
NQ = 4
QN = NP // NQ
CAP = 10752
CC = 32
CAPC = CAP // CC
SLAB = 48
NSLAB = CAPC // SLAB
ROWS_PER_TILE = NP // NS
QROWS_PER_TILE = QN // NS

_mesh = plsc.VectorSubcoreMesh(
    core_axis_name="c", subcore_axis_name="s", num_cores=NC, num_subcores=NS
)


def _part_body(src_hbm, dst_hbm, hist_hbm, lists_hbm, cnt_hbm,
               src_v, dst_v, hist_v,
               q0s, q0d, q1s, q1d, q2s, q2d, q3s, q3d, cnt_v):
    cid = lax.axis_index("c")
    sid = lax.axis_index("s")
    wid = sid * NC + cid
    pltpu.sync_copy(src_hbm.at[pl.ds(wid * EPW, EPW)], src_v)
    pltpu.sync_copy(dst_hbm.at[pl.ds(wid * EPW, EPW)], dst_v)

    nfill = jnp.full((16,), N, jnp.int32)
    zfill = jnp.zeros((16,), jnp.int32)
    zf32 = jnp.zeros((16,), jnp.float32)

    @pl.loop(0, CAP // 16)
    def _fill(i):
        for ref in (q0s, q1s, q2s, q3s):
            ref[pl.ds(i * 16, 16)] = nfill
        for ref in (q0d, q1d, q2d, q3d):
            ref[pl.ds(i * 16, 16)] = zfill

    @pl.loop(0, NP // 16)
    def _zero(i):
        hist_v[pl.ds(i * 16, 16)] = zf32

    qs = ((q0s, q0d), (q1s, q1d), (q2s, q2d), (q3s, q3d))
    ones = jnp.ones((16,), jnp.float32)

    @pl.loop(0, EPW // 16, init_carry=(0, 0, 0, 0))
    def _part(i, carry):
        s16 = src_v[pl.ds(i * 16, 16)]
        d16 = dst_v[pl.ds(i * 16, 16)]
        plsc.addupdate_scatter(hist_v, [d16], ones)
        qid = lax.div(d16, QN)
        new = []
        for q in range(NQ):
            m = qid == q
            ns = lax.reduce_max(plsc.all_reduce_population_count(m), (0,))
            c = carry[q]
            plsc.store_compressed(qs[q][0].at[pl.ds(c, 16)], s16, mask=m)
            plsc.store_compressed(qs[q][1].at[pl.ds(c, 16)], d16 - q * QN, mask=m)
            new.append(c + ns)
        return tuple(new)

    iota = lax.iota(jnp.int32, 16)
    v = jnp.zeros((16,), jnp.int32)
    for q in range(NQ):
        nch = lax.div(_part[q] + CC - 1, CC)
        v = jnp.where(iota == q, nch, v)
    cnt_v[...] = v
    pltpu.sync_copy(cnt_v, cnt_hbm.at[wid])
    pltpu.sync_copy(hist_v, hist_hbm.at[wid])
    for q in range(NQ):
        pltpu.sync_copy(qs[q][0], lists_hbm.at[wid, q, 0])
        pltpu.sync_copy(qs[q][1], lists_hbm.at[wid, q, 1])


_part_call = functools.partial(
    pl.kernel,
    out_type=(
        jax.ShapeDtypeStruct((NW, NP), jnp.float32),
        jax.ShapeDtypeStruct((NW, NQ, 2, CAP), jnp.int32),
        jax.ShapeDtypeStruct((NW, 16), jnp.int32),
    ),
    mesh=_mesh,
    compiler_params=pltpu.CompilerParams(needs_layout_passes=False),
    scratch_types=[
        pltpu.VMEM((EPW,), jnp.int32),
        pltpu.VMEM((EPW,), jnp.int32),
        pltpu.VMEM((NP,), jnp.float32),
    ] + [pltpu.VMEM((CAP,), jnp.int32)] * 8 + [pltpu.VMEM((16,), jnp.int32)],
)(_part_body)


def _disg_kernel(hist_ref, x_ref, w1_ref, b1_ref, g_ref):
    bi = pl.program_id(0)
    deg = jnp.sum(hist_ref[...], axis=0) + 1.0
    dis = lax.rsqrt(deg)
    h = jnp.dot(x_ref[...], w1_ref[...], preferred_element_type=jnp.float32)
    h = h + b1_ref[...]
    rows = lax.broadcasted_iota(jnp.int32, (RB, 1), 0) + bi * RB
    g_ref[...] = jnp.where(rows < N, h * dis[:, None], 0.0)


_disg_call = pl.pallas_call(
    _disg_kernel,
    grid=(NB,),
    in_specs=[
        pl.BlockSpec((NW, RB), lambda i: (0, i)),
        pl.BlockSpec((RB, D), lambda i: (i, 0)),
        pl.BlockSpec((D, H), lambda i: (0, 0)),
        pl.BlockSpec((1, H), lambda i: (0, 0)),
    ],
    out_specs=pl.BlockSpec((RB, H), lambda i: (i, 0)),
    out_shape=jax.ShapeDtypeStruct((NP, H), jnp.float32),
)


def _agg_body(g_hbm, lists_hbm, cnt_hbm, out_hbm,
              idx_s, idx_d, buf0, buf1, buf2, cnt_v, g_sp, aggq,
              sg0, sg1, sg2, ss0, ss1, ss2):
    cid = lax.axis_index("c")
    sid = lax.axis_index("s")

    pltpu.sync_copy(
        g_hbm.at[pl.ds(sid * ROWS_PER_TILE, ROWS_PER_TILE)],
        g_sp.at[pl.ds(sid * ROWS_PER_TILE, ROWS_PER_TILE)],
    )

    iota = lax.iota(jnp.int32, 16)
    bufs = ((buf0, sg0, ss0), (buf1, sg1, ss1), (buf2, sg2, ss2))
    gsrc = (g_hbm, g_sp, g_sp)

    for pq in range(2):
        q = cid * 2 + pq

        @pl.loop(0, CC)
        def _zrow(i):
            for v in range(H // 16):
                buf0[i, pl.ds(v * 16, 16)] = jnp.zeros((16,), jnp.float32)

        base_row = sid * QROWS_PER_TILE
        for zo in range(QROWS_PER_TILE // CC):
            pltpu.sync_copy(buf0, aggq.at[pl.ds(base_row + zo * CC, CC)])
        if QROWS_PER_TILE % CC:
            pltpu.sync_copy(
                buf0.at[pl.ds(0, QROWS_PER_TILE % CC)],
                aggq.at[pl.ds(base_row + (QROWS_PER_TILE // CC) * CC,
                              QROWS_PER_TILE % CC)],
            )
        plsc.subcore_barrier()

        for r in range(2):
            w = sid * 2 + r
            pltpu.sync_copy(cnt_hbm.at[w], cnt_v)
            nch = lax.reduce_max(jnp.where(iota == q, cnt_v[...], 0), (0,))

            for si in range(NSLAB):
                base = si * SLAB

                @pl.when(base < nch)
                def _slab():
                    pltpu.sync_copy(lists_hbm.at[w, q, 0, pl.ds(base, SLAB)], idx_s)
                    pltpu.sync_copy(lists_hbm.at[w, q, 1, pl.ds(base, SLAB)], idx_d)

                    for b in range(3):

                        @pl.when(base + b < nch)
                        def _pro():
                            pltpu.async_copy(
                                gsrc[b].at[idx_s.at[b]], bufs[b][0], bufs[b][1])

                    @pl.loop(0, SLAB - 3, step=3)
                    def _main(j0):
                        for b in range(3):
                            j = j0 + b

                            @pl.when(base + j < nch)
                            def _do():
                                pltpu.make_async_copy(
                                    gsrc[b].at[idx_s.at[j]], bufs[b][0], bufs[b][1]
                                ).wait()
                                pltpu.async_copy(
                                    bufs[b][0], aggq.at[idx_d.at[j]],
                                    bufs[b][2], add=True)

                            @pl.when(base + j + 3 < nch)
                            def _pre():
                                pltpu.make_async_copy(
                                    bufs[b][0], aggq.at[idx_d.at[j]], bufs[b][2]
                                ).wait()
                                pltpu.async_copy(
                                    gsrc[b].at[idx_s.at[j + 3]], bufs[b][0], bufs[b][1])

                    for b in range(3):
                        j = SLAB - 3 + b

                        @pl.when(base + j < nch)
                        def _epi():
                            pltpu.make_async_copy(
                                gsrc[b].at[idx_s.at[j]], bufs[b][0], bufs[b][1]
                            ).wait()
                            pltpu.async_copy(
                                bufs[b][0], aggq.at[idx_d.at[j]],
                                bufs[b][2], add=True)

                    nbc = jnp.minimum(nch - base, SLAB)
                    for b in range(3):
                        jb = nbc - 3 + lax.rem(lax.rem(b - nbc + 3, 3) + 3, 3)

                        @pl.when(jb >= 0)
                        def _drain():
                            pltpu.make_async_copy(
                                bufs[b][0], aggq.at[idx_d.at[jb]], bufs[b][2]
                            ).wait()

        plsc.subcore_barrier()
        pltpu.sync_copy(
            aggq.at[pl.ds(base_row, QROWS_PER_TILE)],
            out_hbm.at[pl.ds(q * QN + base_row, QROWS_PER_TILE)],
        )


_agg_call = functools.partial(
    pl.kernel,
    out_type=jax.ShapeDtypeStruct((NP, H), jnp.float32),
    mesh=_mesh,
    compiler_params=pltpu.CompilerParams(needs_layout_passes=False),
    scratch_types=[
        pltpu.VMEM((SLAB, CC), jnp.int32),
        pltpu.VMEM((SLAB, CC), jnp.int32),
        pltpu.VMEM((CC, H), jnp.float32),
        pltpu.VMEM((CC, H), jnp.float32),
        pltpu.VMEM((CC, H), jnp.float32),
        pltpu.VMEM((16,), jnp.int32),
        pltpu.VMEM_SHARED((NP, H), jnp.float32),
        pltpu.VMEM_SHARED((QN, H), jnp.float32),
        pltpu.SemaphoreType.DMA,
        pltpu.SemaphoreType.DMA,
        pltpu.SemaphoreType.DMA,
        pltpu.SemaphoreType.DMA,
        pltpu.SemaphoreType.DMA,
        pltpu.SemaphoreType.DMA,
    ],
)(_agg_body)


def _out_kernel(agg_ref, g_ref, hist_ref, w2t_ref, b2_ref, out_ref):
    deg = jnp.sum(hist_ref[...], axis=0) + 1.0
    dis = lax.rsqrt(deg)
    t = (agg_ref[...] + g_ref[...]) * dis[:, None]
    t = jnp.maximum(t, 0.0)
    out_ref[...] = jnp.sum(t * w2t_ref[...], axis=1, keepdims=True) + b2_ref[0, 0]


_out_call = pl.pallas_call(
    _out_kernel,
    grid=(NB,),
    in_specs=[
        pl.BlockSpec((RB, H), lambda i: (i, 0)),
        pl.BlockSpec((RB, H), lambda i: (i, 0)),
        pl.BlockSpec((NW, RB), lambda i: (0, i)),
        pl.BlockSpec((1, H), lambda i: (0, 0)),
        pl.BlockSpec((1, 1), lambda i: (0, 0)),
    ],
    out_specs=pl.BlockSpec((RB, 1), lambda i: (i, 0)),
    out_shape=jax.ShapeDtypeStruct((NP, 1), jnp.float32),
)


def kernel(x, edge_index, W1, b1, W2, b2):
    src = edge_index[0].astype(jnp.int32)
    dst = edge_index[1].astype(jnp.int32)
    xp = jnp.pad(x, ((0, NP - N), (0, 0)))

    hist, lists, counts = _part_call(src, dst)
    g = _disg_call(hist, xp, W1, b1.reshape(1, H))
    lists5 = lists.reshape(NW, NQ, 2, CAPC, CC)
    agg = _agg_call(g, lists5, counts)
    outp = _out_call(agg, g, hist, W2.reshape(1, H), b2.reshape(1, 1))
    return outp[:N]


# speedup vs baseline: 1.1126x; 1.1126x over previous
import functools

import jax
import jax.numpy as jnp
from jax import lax
from jax.experimental import pallas as pl
from jax.experimental.pallas import tpu as pltpu
from jax.experimental.pallas import tpu_sc as plsc

N = 10000
E = 320000
D = 128
H = 128

NC = 2
NS = 16
NW = NC * NS

NP = 10240
RB = 128
NB = NP // RB
EPW = E // NW
NQ = 4
QN = NP // NQ
CAP = 10240
CC = 64
CAPC = CAP // CC
SLAB = 40
NSLAB = CAPC // SLAB
ROWS_PER_TILE = NP // NS
QROWS_PER_TILE = QN // NS

_mesh = plsc.VectorSubcoreMesh(
    core_axis_name="c", subcore_axis_name="s", num_cores=NC, num_subcores=NS
)


def _part_body(ei_hbm, hist_hbm, lists_hbm, cnt_hbm,
               src_v, dst_v, hist_v,
               q0s, q0d, q1s, q1d, q2s, q2d, q3s, q3d, cnt_v):
    cid = lax.axis_index("c")
    sid = lax.axis_index("s")
    wid = sid * NC + cid
    pltpu.sync_copy(ei_hbm.at[pl.ds(wid * EPW, EPW)], src_v)
    pltpu.sync_copy(ei_hbm.at[pl.ds(E + wid * EPW, EPW)], dst_v)

    nfill = jnp.full((16,), N, jnp.int32)
    zfill = jnp.zeros((16,), jnp.int32)
    zf32 = jnp.zeros((16,), jnp.float32)

    @pl.loop(0, CAP // 16)
    def _fill(i):
        for ref in (q0s, q1s, q2s, q3s):
            ref[pl.ds(i * 16, 16)] = nfill
        for ref in (q0d, q1d, q2d, q3d):
            ref[pl.ds(i * 16, 16)] = zfill

    @pl.loop(0, NP // 16)
    def _zero(i):
        hist_v[pl.ds(i * 16, 16)] = zf32

    qs = ((q0s, q0d), (q1s, q1d), (q2s, q2d), (q3s, q3d))
    ones = jnp.ones((16,), jnp.float32)

    @pl.loop(0, EPW // 16, init_carry=(0, 0, 0, 0))
    def _part(i, carry):
        s16 = src_v[pl.ds(i * 16, 16)]
        d16 = dst_v[pl.ds(i * 16, 16)]
        plsc.addupdate_scatter(hist_v, [d16], ones)
        qid = lax.div(d16, QN)
        new = []
        for q in range(NQ):
            m = qid == q
            ns = lax.reduce_max(plsc.all_reduce_population_count(m), (0,))
            c = carry[q]
            plsc.store_compressed(qs[q][0].at[pl.ds(c, 16)], s16, mask=m)
            plsc.store_compressed(qs[q][1].at[pl.ds(c, 16)], d16 - q * QN, mask=m)
            new.append(c + ns)
        return tuple(new)

    iota = lax.iota(jnp.int32, 16)
    v = jnp.zeros((16,), jnp.int32)
    for q in range(NQ):
        nch = lax.div(_part[q] + CC - 1, CC)
        v = jnp.where(iota == q, nch, v)
    cnt_v[...] = v
    pltpu.sync_copy(cnt_v, cnt_hbm.at[wid])
    pltpu.sync_copy(hist_v, hist_hbm.at[wid])
    for q in range(NQ):
        pltpu.sync_copy(qs[q][0], lists_hbm.at[wid, q, 0])
        pltpu.sync_copy(qs[q][1], lists_hbm.at[wid, q, 1])


_part_call = functools.partial(
    pl.kernel,
    out_type=(
        jax.ShapeDtypeStruct((NW, NP), jnp.float32),
        jax.ShapeDtypeStruct((NW, NQ, 2, CAP), jnp.int32),
        jax.ShapeDtypeStruct((NW, 16), jnp.int32),
    ),
    mesh=_mesh,
    compiler_params=pltpu.CompilerParams(needs_layout_passes=False),
    scratch_types=[
        pltpu.VMEM((EPW,), jnp.int32),
        pltpu.VMEM((EPW,), jnp.int32),
        pltpu.VMEM((NP,), jnp.float32),
    ] + [pltpu.VMEM((CAP,), jnp.int32)] * 8 + [pltpu.VMEM((16,), jnp.int32)],
)(_part_body)


def _disg_kernel(hist_ref, x_ref, w1_ref, b1_ref, g_ref):
    bi = pl.program_id(0)
    deg = jnp.sum(hist_ref[...], axis=0) + 1.0
    dis = lax.rsqrt(deg)
    h = jnp.dot(x_ref[...], w1_ref[...], preferred_element_type=jnp.float32)
    h = h + b1_ref[...]
    rows = lax.broadcasted_iota(jnp.int32, (RB, 1), 0) + bi * RB
    g_ref[...] = jnp.where(rows < N, h * dis[:, None], 0.0)


_disg_call = pl.pallas_call(
    _disg_kernel,
    grid=(NB,),
    in_specs=[
        pl.BlockSpec((NW, RB), lambda i: (0, i)),
        pl.BlockSpec((RB, D), lambda i: (jnp.minimum(i, N // RB), 0)),
        pl.BlockSpec((D, H), lambda i: (0, 0)),
        pl.BlockSpec((1, H), lambda i: (0, 0)),
    ],
    out_specs=pl.BlockSpec((RB, H), lambda i: (i, 0)),
    out_shape=jax.ShapeDtypeStruct((NP, H), jnp.float32),
)


def _agg_body(g_hbm, lists_hbm, cnt_hbm, out_hbm,
              idx_s, idx_d, buf0, buf1, cnt_v, g_sp, aggq, sg0, sg1):
    cid = lax.axis_index("c")
    sid = lax.axis_index("s")

    pltpu.sync_copy(
        g_hbm.at[pl.ds(sid * ROWS_PER_TILE, ROWS_PER_TILE)],
        g_sp.at[pl.ds(sid * ROWS_PER_TILE, ROWS_PER_TILE)],
    )

    iota = lax.iota(jnp.int32, 16)
    bufs = ((buf0, sg0), (buf1, sg1))

    for pq in range(2):
        q = cid * 2 + pq

        @pl.loop(0, CC)
        def _zrow(i):
            for v in range(H // 16):
                buf0[i, pl.ds(v * 16, 16)] = jnp.zeros((16,), jnp.float32)

        base_row = sid * QROWS_PER_TILE
        pltpu.sync_copy(buf0, aggq.at[pl.ds(base_row, CC)])
        pltpu.sync_copy(buf0, aggq.at[pl.ds(base_row + CC, CC)])
        pltpu.sync_copy(buf0.at[pl.ds(0, 32)], aggq.at[pl.ds(base_row + 2 * CC, 32)])
        plsc.subcore_barrier()

        for r in range(2):
            w = sid * 2 + r
            pltpu.sync_copy(cnt_hbm.at[w], cnt_v)
            nch = lax.reduce_max(jnp.where(iota == q, cnt_v[...], 0), (0,))

            for si in range(NSLAB):
                base = si * SLAB

                @pl.when(base < nch)
                def _slab():
                    pltpu.sync_copy(lists_hbm.at[w, q, 0, pl.ds(base, SLAB)], idx_s)
                    pltpu.sync_copy(lists_hbm.at[w, q, 1, pl.ds(base, SLAB)], idx_d)

                    for b in range(2):

                        @pl.when(base + b < nch)
                        def _pro():
                            pltpu.async_copy(
                                g_sp.at[idx_s.at[b]], bufs[b][0], bufs[b][1])

                    @pl.loop(0, SLAB - 2, step=2)
                    def _main(j0):
                        for b in range(2):
                            j = j0 + b

                            @pl.when(base + j < nch)
                            def _do():
                                pltpu.make_async_copy(
                                    g_sp.at[idx_s.at[j]], bufs[b][0], bufs[b][1]
                                ).wait()
                                pltpu.sync_copy(
                                    bufs[b][0], aggq.at[idx_d.at[j]], add=True)

                            @pl.when(base + j + 2 < nch)
                            def _pre():
                                pltpu.async_copy(
                                    g_sp.at[idx_s.at[j + 2]], bufs[b][0], bufs[b][1])

                    for b in range(2):
                        j = SLAB - 2 + b

                        @pl.when(base + j < nch)
                        def _epi():
                            pltpu.make_async_copy(
                                g_sp.at[idx_s.at[j]], bufs[b][0], bufs[b][1]
                            ).wait()
                            pltpu.sync_copy(
                                bufs[b][0], aggq.at[idx_d.at[j]], add=True)

        plsc.subcore_barrier()
        pltpu.sync_copy(
            aggq.at[pl.ds(base_row, QROWS_PER_TILE)],
            out_hbm.at[pl.ds(q * QN + base_row, QROWS_PER_TILE)],
        )


_agg_call = functools.partial(
    pl.kernel,
    out_type=jax.ShapeDtypeStruct((NP, H), jnp.float32),
    mesh=_mesh,
    compiler_params=pltpu.CompilerParams(needs_layout_passes=False),
    scratch_types=[
        pltpu.VMEM((SLAB, CC), jnp.int32),
        pltpu.VMEM((SLAB, CC), jnp.int32),
        pltpu.VMEM((CC, H), jnp.float32),
        pltpu.VMEM((CC, H), jnp.float32),
        pltpu.VMEM((16,), jnp.int32),
        pltpu.VMEM_SHARED((NP, H), jnp.float32),
        pltpu.VMEM_SHARED((QN, H), jnp.float32),
        pltpu.SemaphoreType.DMA,
        pltpu.SemaphoreType.DMA,
    ],
)(_agg_body)


def _out_kernel(agg_ref, g_ref, hist_ref, w2t_ref, b2_ref, out_ref):
    deg = jnp.sum(hist_ref[...], axis=0) + 1.0
    dis = lax.rsqrt(deg)
    t = (agg_ref[...] + g_ref[...]) * dis[:, None]
    t = jnp.maximum(t, 0.0)
    out_ref[...] = jnp.sum(t * w2t_ref[...], axis=1, keepdims=True) + b2_ref[0, 0]


_out_call = pl.pallas_call(
    _out_kernel,
    grid=(N // RB + 1,),
    in_specs=[
        pl.BlockSpec((RB, H), lambda i: (i, 0)),
        pl.BlockSpec((RB, H), lambda i: (i, 0)),
        pl.BlockSpec((NW, RB), lambda i: (0, i)),
        pl.BlockSpec((1, H), lambda i: (0, 0)),
        pl.BlockSpec((1, 1), lambda i: (0, 0)),
    ],
    out_specs=pl.BlockSpec((RB, 1), lambda i: (i, 0)),
    out_shape=jax.ShapeDtypeStruct((N, 1), jnp.float32),
)


def kernel(x, edge_index, W1, b1, W2, b2):
    ei = edge_index.astype(jnp.int32).reshape(2 * E)

    hist, lists, counts = _part_call(ei)
    g = _disg_call(hist, x, W1, b1.reshape(1, H))
    lists5 = lists.reshape(NW, NQ, 2, CAPC, CC)
    agg = _agg_call(g, lists5, counts)
    return _out_call(agg, g, hist, W2.reshape(1, H), b2.reshape(1, 1))
